# baseline (speedup 1.0000x reference)
import jax
import jax.numpy as jnp
from jax.experimental import pallas as pl
from jax.experimental.pallas import tpu as pltpu

N_EXPERT = 8
TOPK = 2
D_MODEL = 1024
D_FF = 1024
T_TOK = 2048
G = 256
SEG = 128
NT = T_TOK * TOPK // G + N_EXPERT
P = NT * G


def _nt_dot(a, b):
    return jax.lax.dot_general(a, b, (((1,), (1,)), ((), ())),
                               preferred_element_type=jnp.float32)


def _dot(a, b):
    return jax.lax.dot_general(a, b, (((1,), (0,)), ((), ())),
                               preferred_element_type=jnp.float32)


def _route_body(x_ref, gw_ref, pos_ref, prob_ref, te_ref):
    x = x_ref[...]
    gw = gw_ref[...]
    router = _nt_dot(x, gw)
    iota8 = jax.lax.broadcasted_iota(jnp.int32, router.shape, 1)
    m0 = jnp.max(router, axis=1, keepdims=True)
    i0 = jnp.min(jnp.where(router == m0, iota8, N_EXPERT), axis=1, keepdims=True)
    masked = jnp.where(iota8 == i0, -jnp.inf, router)
    m1 = jnp.max(masked, axis=1, keepdims=True)
    i1 = jnp.min(jnp.where(masked == m1, iota8, N_EXPERT), axis=1, keepdims=True)
    e1 = jnp.exp(m1 - m0)
    denom = 1.0 + e1
    p0 = 1.0 / denom
    p1 = e1 / denom

    oh0 = (iota8 == i0).astype(jnp.float32)
    oh1 = (iota8 == i1).astype(jnp.float32)

    r_iota = jax.lax.broadcasted_iota(jnp.int32, (SEG, SEG), 0)
    c_iota = jax.lax.broadcasted_iota(jnp.int32, (SEG, SEG), 1)
    lstrict = (c_iota < r_iota).astype(jnp.float32)

    def seg_ranks(oh, off):
        ranks = []
        for b in range(T_TOK // SEG):
            seg = oh[b * SEG:(b + 1) * SEG, :]
            ranks.append(_dot(lstrict, seg) + off)
            off = off + jnp.sum(seg, axis=0, keepdims=True)
        return jnp.concatenate(ranks, axis=0), off

    zero8 = jnp.zeros((1, N_EXPERT), jnp.float32)
    rank0, cnt0 = seg_ranks(oh0, zero8)
    rank1, cnt = seg_ranks(oh1, cnt0)

    cnti = cnt.astype(jnp.int32)
    padded = ((cnti + (G - 1)) // G) * G
    e_iota_r = jax.lax.broadcasted_iota(jnp.int32, (N_EXPERT, N_EXPERT), 0)
    e_iota_c = jax.lax.broadcasted_iota(jnp.int32, (N_EXPERT, N_EXPERT), 1)
    u8strict = (e_iota_r < e_iota_c).astype(jnp.float32)
    start = _dot(padded.astype(jnp.float32), u8strict)

    pos0 = jnp.sum(oh0 * (start + rank0), axis=1, keepdims=True)
    pos1 = jnp.sum(oh1 * (start + rank1), axis=1, keepdims=True)
    pos_ref[...] = jnp.concatenate([pos0, pos1], axis=1).astype(jnp.int32)
    prob_ref[...] = jnp.concatenate([p0, p1], axis=1)

    ident8 = (e_iota_r == e_iota_c).astype(jnp.float32)
    ts_col = _nt_dot(ident8, start * (1.0 / G))
    t_iota = jax.lax.broadcasted_iota(jnp.int32, (N_EXPERT, SEG), 1).astype(jnp.float32)
    ind = (t_iota >= ts_col).astype(jnp.float32)
    sel = (jax.lax.broadcasted_iota(jnp.int32, (1, N_EXPERT), 1) >= 1).astype(jnp.float32)
    te_row = _dot(sel, ind)
    te_ref[...] = jnp.broadcast_to(te_row, (N_EXPERT, SEG)).astype(jnp.int32)


def _route(xf, gate_w):
    return pl.pallas_call(
        _route_body,
        out_shape=(
            jax.ShapeDtypeStruct((T_TOK, TOPK), jnp.int32),
            jax.ShapeDtypeStruct((T_TOK, TOPK), jnp.float32),
            jax.ShapeDtypeStruct((N_EXPERT, SEG), jnp.int32),
        ),
    )(xf, gate_w)


def _mlp_body(te_ref, xs_ref, w1_ref, w2_ref, w3_ref, o_ref):
    xb = xs_ref[...]
    w1 = w1_ref[0]
    w2 = w2_ref[0]
    w3 = w3_ref[0]
    h1 = _nt_dot(xb, w1)
    h2 = _nt_dot(xb, w2)
    h = (h1 * (1.0 / (1.0 + jnp.exp(-h1)))) * h2
    o_ref[...] = _nt_dot(h, w3)


def _grouped_mlp(te, xs, w1, w2, w3):
    grid_spec = pltpu.PrefetchScalarGridSpec(
        num_scalar_prefetch=1,
        grid=(NT,),
        in_specs=[
            pl.BlockSpec((G, D_MODEL), lambda i, te: (i, 0)),
            pl.BlockSpec((1, D_FF, D_MODEL), lambda i, te: (te[i], 0, 0)),
            pl.BlockSpec((1, D_FF, D_MODEL), lambda i, te: (te[i], 0, 0)),
            pl.BlockSpec((1, D_MODEL, D_FF), lambda i, te: (te[i], 0, 0)),
        ],
        out_specs=pl.BlockSpec((G, D_MODEL), lambda i, te: (i, 0)),
    )
    return pl.pallas_call(
        _mlp_body,
        grid_spec=grid_spec,
        out_shape=jax.ShapeDtypeStruct((P, D_MODEL), jnp.float32),
        compiler_params=pltpu.CompilerParams(
            dimension_semantics=("arbitrary",),
        ),
    )(te, xs, w1, w2, w3)


def kernel(x, gate_w, w1, w2, w3):
    Bq, Tq, C = x.shape
    xf = x.reshape(Tq, C)
    pos, prob, te_blk = _route(xf, gate_w)
    te = te_blk[0, :NT]

    pos_cat = jnp.concatenate([pos[:, 0], pos[:, 1]])
    xs = jnp.zeros((P, C), jnp.float32).at[pos_cat].set(
        jnp.concatenate([xf, xf], axis=0))

    out_s = _grouped_mlp(te, xs, w1, w2, w3)

    y = (prob[:, 0:1] * out_s[pos[:, 0]] +
         prob[:, 1:2] * out_s[pos[:, 1]])
    return y.reshape(Bq, Tq, C)


# speedup vs baseline: 1.0979x; 1.0979x over previous
import functools

import jax
import jax.numpy as jnp
from jax import lax
from jax.experimental import pallas as pl
from jax.experimental.pallas import tpu as pltpu
from jax.experimental.pallas import tpu_sc as plsc

N_EXPERT = 8
TOPK = 2
D_MODEL = 1024
D_FF = 1024
T_TOK = 2048
G = 256
SEG = 128
NT = T_TOK * TOPK // G + N_EXPERT
P = NT * G


def _nt_dot(a, b):
    return jax.lax.dot_general(a, b, (((1,), (1,)), ((), ())),
                               preferred_element_type=jnp.float32)


def _dot(a, b):
    return jax.lax.dot_general(a, b, (((1,), (0,)), ((), ())),
                               preferred_element_type=jnp.float32)


def _route_body(x_ref, gw_ref, pos_ref, prob_ref, te_ref):
    x = x_ref[...]
    gw = gw_ref[...]
    router = _nt_dot(x, gw)
    iota8 = jax.lax.broadcasted_iota(jnp.int32, router.shape, 1)
    m0 = jnp.max(router, axis=1, keepdims=True)
    i0 = jnp.min(jnp.where(router == m0, iota8, N_EXPERT), axis=1, keepdims=True)
    masked = jnp.where(iota8 == i0, -jnp.inf, router)
    m1 = jnp.max(masked, axis=1, keepdims=True)
    i1 = jnp.min(jnp.where(masked == m1, iota8, N_EXPERT), axis=1, keepdims=True)
    e1 = jnp.exp(m1 - m0)
    denom = 1.0 + e1
    p0 = 1.0 / denom
    p1 = e1 / denom

    oh0 = (iota8 == i0).astype(jnp.float32)
    oh1 = (iota8 == i1).astype(jnp.float32)

    r_iota = jax.lax.broadcasted_iota(jnp.int32, (SEG, SEG), 0)
    c_iota = jax.lax.broadcasted_iota(jnp.int32, (SEG, SEG), 1)
    lstrict = (c_iota < r_iota).astype(jnp.float32)

    def seg_ranks(oh, off):
        ranks = []
        for b in range(T_TOK // SEG):
            seg = oh[b * SEG:(b + 1) * SEG, :]
            ranks.append(_dot(lstrict, seg) + off)
            off = off + jnp.sum(seg, axis=0, keepdims=True)
        return jnp.concatenate(ranks, axis=0), off

    zero8 = jnp.zeros((1, N_EXPERT), jnp.float32)
    rank0, cnt0 = seg_ranks(oh0, zero8)
    rank1, cnt = seg_ranks(oh1, cnt0)

    cnti = cnt.astype(jnp.int32)
    padded = ((cnti + (G - 1)) // G) * G
    e_iota_r = jax.lax.broadcasted_iota(jnp.int32, (N_EXPERT, N_EXPERT), 0)
    e_iota_c = jax.lax.broadcasted_iota(jnp.int32, (N_EXPERT, N_EXPERT), 1)
    u8strict = (e_iota_r < e_iota_c).astype(jnp.float32)
    start = _dot(padded.astype(jnp.float32), u8strict)

    pos0 = jnp.sum(oh0 * (start + rank0), axis=1, keepdims=True)
    pos1 = jnp.sum(oh1 * (start + rank1), axis=1, keepdims=True)
    pos_ref[...] = jnp.concatenate([pos0, pos1], axis=1).astype(jnp.int32)
    prob_ref[...] = jnp.concatenate([p0, p1], axis=1)

    ident8 = (e_iota_r == e_iota_c).astype(jnp.float32)
    ts_col = _nt_dot(ident8, start * (1.0 / G))
    t_iota = jax.lax.broadcasted_iota(jnp.int32, (N_EXPERT, SEG), 1).astype(jnp.float32)
    ind = (t_iota >= ts_col).astype(jnp.float32)
    sel = (jax.lax.broadcasted_iota(jnp.int32, (1, N_EXPERT), 1) >= 1).astype(jnp.float32)
    te_row = _dot(sel, ind)
    te_ref[...] = jnp.broadcast_to(te_row, (N_EXPERT, SEG)).astype(jnp.int32)


def _route(xf, gate_w):
    return pl.pallas_call(
        _route_body,
        out_shape=(
            jax.ShapeDtypeStruct((T_TOK, TOPK), jnp.int32),
            jax.ShapeDtypeStruct((T_TOK, TOPK), jnp.float32),
            jax.ShapeDtypeStruct((N_EXPERT, SEG), jnp.int32),
        ),
    )(xf, gate_w)


def _mlp_body(te_ref, xs_ref, w1_ref, w2_ref, w3_ref, o_ref):
    xb = xs_ref[...]
    w1 = w1_ref[0]
    w2 = w2_ref[0]
    w3 = w3_ref[0]
    h1 = _nt_dot(xb, w1)
    h2 = _nt_dot(xb, w2)
    h = (h1 * (1.0 / (1.0 + jnp.exp(-h1)))) * h2
    o_ref[...] = _nt_dot(h, w3)


def _grouped_mlp(te, xs, w1, w2, w3):
    grid_spec = pltpu.PrefetchScalarGridSpec(
        num_scalar_prefetch=1,
        grid=(NT,),
        in_specs=[
            pl.BlockSpec((G, D_MODEL), lambda i, te: (i, 0)),
            pl.BlockSpec((1, D_FF, D_MODEL), lambda i, te: (te[i], 0, 0)),
            pl.BlockSpec((1, D_FF, D_MODEL), lambda i, te: (te[i], 0, 0)),
            pl.BlockSpec((1, D_MODEL, D_FF), lambda i, te: (te[i], 0, 0)),
        ],
        out_specs=pl.BlockSpec((G, D_MODEL), lambda i, te: (i, 0)),
    )
    return pl.pallas_call(
        _mlp_body,
        grid_spec=grid_spec,
        out_shape=jax.ShapeDtypeStruct((P, D_MODEL), jnp.float32),
        compiler_params=pltpu.CompilerParams(
            dimension_semantics=("arbitrary",),
        ),
    )(te, xs, w1, w2, w3)


_SC_MESH = plsc.VectorSubcoreMesh(core_axis_name="c", subcore_axis_name="s")
_NC = 2
_NS = 16
_NW = _NC * _NS


def _dispatch_body(x_hbm, pos_hbm, xs_hbm, idx_v, buf_v, sem):
    wid = lax.axis_index("s") * _NC + lax.axis_index("c")
    tb = (wid % 16) * 128
    ab = wid * 128
    for c in range(2):
        pltpu.sync_copy(pos_hbm.at[pl.ds(ab + c * 64, 64)], idx_v)
        pltpu.sync_copy(x_hbm.at[pl.ds(tb + c * 64, 64), :], buf_v)
        pltpu.async_copy(buf_v, xs_hbm.at[idx_v], sem).wait()


@functools.partial(
    pl.kernel,
    out_type=jax.ShapeDtypeStruct((P, D_MODEL), jnp.float32),
    mesh=_SC_MESH,
    scratch_types=[
        pltpu.VMEM((64,), jnp.int32),
        pltpu.VMEM((64, D_MODEL), jnp.float32),
        pltpu.SemaphoreType.DMA,
    ],
)
def _dispatch(x_hbm, pos_hbm, xs_hbm, idx_v, buf_v, sem):
    _dispatch_body(x_hbm, pos_hbm, xs_hbm, idx_v, buf_v, sem)


def _bcast_lane(vec16, lane):
    idx = jnp.full((16, 1), lane, jnp.int32)
    dn = lax.GatherDimensionNumbers(
        offset_dims=(), collapsed_slice_dims=(0,), start_index_map=(0,))
    return lax.gather(vec16, idx, dn, (1,),
                      mode=lax.GatherScatterMode.PROMISE_IN_BOUNDS)


def _combine_body(os_hbm, pos0_hbm, pos1_hbm, pr0_hbm, pr1_hbm, y_hbm,
                  idx0_v, idx1_v, pr0_v, pr1_v, r0_v, r1_v, y_v, sem):
    wid = lax.axis_index("s") * _NC + lax.axis_index("c")
    tb = wid * 64
    for c in range(2):
        base = tb + c * 32
        pltpu.sync_copy(pos0_hbm.at[pl.ds(base, 32)], idx0_v)
        pltpu.sync_copy(pos1_hbm.at[pl.ds(base, 32)], idx1_v)
        pltpu.sync_copy(pr0_hbm.at[pl.ds(base, 32)], pr0_v)
        pltpu.sync_copy(pr1_hbm.at[pl.ds(base, 32)], pr1_v)
        pltpu.async_copy(os_hbm.at[idx0_v], r0_v, sem).wait()
        pltpu.async_copy(os_hbm.at[idx1_v], r1_v, sem).wait()

        for half in range(2):
            ch0 = pr0_v[pl.ds(half * 16, 16)]
            ch1 = pr1_v[pl.ds(half * 16, 16)]

            def row(r, _, ch0=ch0, ch1=ch1, half=half):
                i = half * 16 + r
                p0 = _bcast_lane(ch0, r)
                p1 = _bcast_lane(ch1, r)
                for cc in range(D_MODEL // 16):
                    sl = pl.ds(cc * 16, 16)
                    y_v[i, sl] = p0 * r0_v[i, sl] + p1 * r1_v[i, sl]
                return 0

            lax.fori_loop(0, 16, row, 0)
        pltpu.sync_copy(y_v, y_hbm.at[pl.ds(base, 32), :])


@functools.partial(
    pl.kernel,
    out_type=jax.ShapeDtypeStruct((T_TOK, D_MODEL), jnp.float32),
    mesh=_SC_MESH,
    scratch_types=[
        pltpu.VMEM((32,), jnp.int32),
        pltpu.VMEM((32,), jnp.int32),
        pltpu.VMEM((32,), jnp.float32),
        pltpu.VMEM((32,), jnp.float32),
        pltpu.VMEM((32, D_MODEL), jnp.float32),
        pltpu.VMEM((32, D_MODEL), jnp.float32),
        pltpu.VMEM((32, D_MODEL), jnp.float32),
        pltpu.SemaphoreType.DMA,
    ],
)
def _combine(os_hbm, pos0_hbm, pos1_hbm, pr0_hbm, pr1_hbm, y_hbm,
             idx0_v, idx1_v, pr0_v, pr1_v, r0_v, r1_v, y_v, sem):
    _combine_body(os_hbm, pos0_hbm, pos1_hbm, pr0_hbm, pr1_hbm, y_hbm,
                  idx0_v, idx1_v, pr0_v, pr1_v, r0_v, r1_v, y_v, sem)


def kernel(x, gate_w, w1, w2, w3):
    Bq, Tq, C = x.shape
    xf = x.reshape(Tq, C)
    pos, prob, te_blk = _route(xf, gate_w)
    te = te_blk[0, :NT]

    pos_cat = jnp.concatenate([pos[:, 0], pos[:, 1]])
    xs = _dispatch(xf, pos_cat)

    out_s = _grouped_mlp(te, xs, w1, w2, w3)

    y = _combine(out_s, pos[:, 0], pos[:, 1], prob[:, 0], prob[:, 1])
    return y.reshape(Bq, Tq, C)


# speedup vs baseline: 1.3086x; 1.1919x over previous
import functools

import jax
import jax.numpy as jnp
from jax import lax
from jax.experimental import pallas as pl
from jax.experimental.pallas import tpu as pltpu
from jax.experimental.pallas import tpu_sc as plsc

N_EXPERT = 8
TOPK = 2
D_MODEL = 1024
D_FF = 1024
T_TOK = 2048
G = 512
SEG = 128
NT = T_TOK * TOPK // G + N_EXPERT
P = NT * G


def _nt_dot(a, b):
    return jax.lax.dot_general(a, b, (((1,), (1,)), ((), ())),
                               preferred_element_type=jnp.float32)


def _dot(a, b):
    return jax.lax.dot_general(a, b, (((1,), (0,)), ((), ())),
                               preferred_element_type=jnp.float32)


def _route_body(x_ref, gw_ref, pos_ref, prob_ref, te_ref):
    x = x_ref[...]
    gw = gw_ref[...]
    router = _nt_dot(x, gw)
    iota8 = jax.lax.broadcasted_iota(jnp.int32, router.shape, 1)
    m0 = jnp.max(router, axis=1, keepdims=True)
    i0 = jnp.min(jnp.where(router == m0, iota8, N_EXPERT), axis=1, keepdims=True)
    masked = jnp.where(iota8 == i0, -jnp.inf, router)
    m1 = jnp.max(masked, axis=1, keepdims=True)
    i1 = jnp.min(jnp.where(masked == m1, iota8, N_EXPERT), axis=1, keepdims=True)
    e1 = jnp.exp(m1 - m0)
    denom = 1.0 + e1
    p0 = 1.0 / denom
    p1 = e1 / denom

    oh0 = (iota8 == i0).astype(jnp.float32)
    oh1 = (iota8 == i1).astype(jnp.float32)

    r_iota = jax.lax.broadcasted_iota(jnp.int32, (SEG, SEG), 0)
    c_iota = jax.lax.broadcasted_iota(jnp.int32, (SEG, SEG), 1)
    lstrict = (c_iota < r_iota).astype(jnp.float32)

    def seg_ranks(oh, off):
        ranks = []
        for b in range(T_TOK // SEG):
            seg = oh[b * SEG:(b + 1) * SEG, :]
            ranks.append(_dot(lstrict, seg) + off)
            off = off + jnp.sum(seg, axis=0, keepdims=True)
        return jnp.concatenate(ranks, axis=0), off

    zero8 = jnp.zeros((1, N_EXPERT), jnp.float32)
    rank0, cnt0 = seg_ranks(oh0, zero8)
    rank1, cnt = seg_ranks(oh1, cnt0)

    cnti = cnt.astype(jnp.int32)
    padded = ((cnti + (G - 1)) // G) * G
    e_iota_r = jax.lax.broadcasted_iota(jnp.int32, (N_EXPERT, N_EXPERT), 0)
    e_iota_c = jax.lax.broadcasted_iota(jnp.int32, (N_EXPERT, N_EXPERT), 1)
    u8strict = (e_iota_r < e_iota_c).astype(jnp.float32)
    start = _dot(padded.astype(jnp.float32), u8strict)

    pos0 = jnp.sum(oh0 * (start + rank0), axis=1, keepdims=True)
    pos1 = jnp.sum(oh1 * (start + rank1), axis=1, keepdims=True)
    pos_ref[...] = jnp.concatenate([pos0, pos1], axis=1).astype(jnp.int32)
    prob_ref[...] = jnp.concatenate([p0, p1], axis=1)

    ident8 = (e_iota_r == e_iota_c).astype(jnp.float32)
    ts_col = _nt_dot(ident8, start * (1.0 / G))
    t_iota = jax.lax.broadcasted_iota(jnp.int32, (N_EXPERT, SEG), 1).astype(jnp.float32)
    ind = (t_iota >= ts_col).astype(jnp.float32)
    sel = (jax.lax.broadcasted_iota(jnp.int32, (1, N_EXPERT), 1) >= 1).astype(jnp.float32)
    te_row = _dot(sel, ind)
    nact = jnp.sum((padded // G).astype(jnp.float32), axis=1, keepdims=True)
    nact_row = jnp.broadcast_to(nact, (1, SEG))
    te_ref[...] = jnp.concatenate(
        [te_row, nact_row] + [te_row] * (N_EXPERT - 2), axis=0).astype(jnp.int32)


def _route(xf, gate_w):
    return pl.pallas_call(
        _route_body,
        out_shape=(
            jax.ShapeDtypeStruct((T_TOK, TOPK), jnp.int32),
            jax.ShapeDtypeStruct((T_TOK, TOPK), jnp.float32),
            jax.ShapeDtypeStruct((N_EXPERT, SEG), jnp.int32),
        ),
    )(xf, gate_w)


def _mlp_body(te_ref, xs_ref, w1_ref, w2_ref, w3_ref, o_ref):
    i = pl.program_id(0)

    @pl.when(i < te_ref[1, 0])
    def _compute():
        xb = xs_ref[...]
        w1 = w1_ref[0]
        w2 = w2_ref[0]
        w3 = w3_ref[0]
        h1 = _nt_dot(xb, w1)
        h2 = _nt_dot(xb, w2)
        h = (h1 * (1.0 / (1.0 + jnp.exp(-h1)))) * h2
        o_ref[...] = _nt_dot(h, w3)


def _grouped_mlp(te2d, xs, w1, w2, w3):
    grid_spec = pltpu.PrefetchScalarGridSpec(
        num_scalar_prefetch=1,
        grid=(NT,),
        in_specs=[
            pl.BlockSpec((G, D_MODEL), lambda i, te: (i, 0)),
            pl.BlockSpec((1, D_FF, D_MODEL), lambda i, te: (te[0, i], 0, 0)),
            pl.BlockSpec((1, D_FF, D_MODEL), lambda i, te: (te[0, i], 0, 0)),
            pl.BlockSpec((1, D_MODEL, D_FF), lambda i, te: (te[0, i], 0, 0)),
        ],
        out_specs=pl.BlockSpec((G, D_MODEL), lambda i, te: (i, 0)),
    )
    return pl.pallas_call(
        _mlp_body,
        grid_spec=grid_spec,
        out_shape=jax.ShapeDtypeStruct((P, D_MODEL), jnp.float32),
        compiler_params=pltpu.CompilerParams(
            dimension_semantics=("arbitrary",),
        ),
    )(te2d, xs, w1, w2, w3)


_NC = 2
_NS = 16
_NW = _NC * _NS


def _dispatch_body(x_hbm, pos_hbm, xs_hbm, idx0_v, idx1_v, idx2_v, idx3_v,
                   buf0_v, buf1_v, semr0, semr1, semw0, semw1):
    wid = lax.axis_index("s") * _NC + lax.axis_index("c")
    tb = (wid % 16) * 128
    ab = wid * 128
    idxs = (idx0_v, idx1_v, idx2_v, idx3_v)
    bufs = (buf0_v, buf1_v)
    semr = (semr0, semr1)
    semw = (semw0, semw1)
    reads = [None] * 4
    writes = [None] * 4
    for c in range(4):
        pltpu.sync_copy(pos_hbm.at[pl.ds(ab + c * 32, 32)], idxs[c])
    for c in range(4):
        b = c % 2
        if c >= 2:
            writes[c - 2].wait()
        reads[c] = pltpu.async_copy(
            x_hbm.at[pl.ds(tb + c * 32, 32), :], bufs[b], semr[b])
        reads[c].wait()
        writes[c] = pltpu.async_copy(bufs[b], xs_hbm.at[idxs[c]], semw[b])
    writes[2].wait()
    writes[3].wait()


@functools.cache
def _make_dispatch():
    mesh = plsc.VectorSubcoreMesh(core_axis_name="c", subcore_axis_name="s")

    @functools.partial(
        pl.kernel,
        out_type=jax.ShapeDtypeStruct((P, D_MODEL), jnp.float32),
        mesh=mesh,
        scratch_types=[
            pltpu.VMEM((32,), jnp.int32),
            pltpu.VMEM((32,), jnp.int32),
            pltpu.VMEM((32,), jnp.int32),
            pltpu.VMEM((32,), jnp.int32),
            pltpu.VMEM((32, D_MODEL), jnp.float32),
            pltpu.VMEM((32, D_MODEL), jnp.float32),
            pltpu.SemaphoreType.DMA,
            pltpu.SemaphoreType.DMA,
            pltpu.SemaphoreType.DMA,
            pltpu.SemaphoreType.DMA,
        ],
    )
    def _dispatch(x_hbm, pos_hbm, xs_hbm, idx0_v, idx1_v, idx2_v, idx3_v,
                  buf0_v, buf1_v, semr0, semr1, semw0, semw1):
        _dispatch_body(x_hbm, pos_hbm, xs_hbm, idx0_v, idx1_v, idx2_v, idx3_v,
                       buf0_v, buf1_v, semr0, semr1, semw0, semw1)

    return _dispatch


def _bcast_lane(vec16, lane):
    idx = jnp.full((16, 1), lane, jnp.int32)
    dn = lax.GatherDimensionNumbers(
        offset_dims=(), collapsed_slice_dims=(0,), start_index_map=(0,))
    return lax.gather(vec16, idx, dn, (1,),
                      mode=lax.GatherScatterMode.PROMISE_IN_BOUNDS)


def _combine_body(os_hbm, pos0_hbm, pos1_hbm, pr0_hbm, pr1_hbm, y_hbm,
                  idx00, idx01, idx10, idx11, pr0_v, pr1_v,
                  r00, r01, r10, r11, y0_v, y1_v,
                  semg0, semg1, semy0, semy1):
    wid = lax.axis_index("s") * _NC + lax.axis_index("c")
    tb = wid * 64
    idx0s = (idx00, idx01)
    idx1s = (idx10, idx11)
    r0s = (r00, r01)
    r1s = (r10, r11)
    ys = (y0_v, y1_v)
    semg = (semg0, semg1)
    semy = (semy0, semy1)
    pltpu.sync_copy(pr0_hbm.at[pl.ds(tb, 64)], pr0_v)
    pltpu.sync_copy(pr1_hbm.at[pl.ds(tb, 64)], pr1_v)
    gathers = [None] * 4
    ywrites = [None] * 4

    def start_chunk(c):
        b = c % 2
        base = tb + c * 16
        pltpu.sync_copy(pos0_hbm.at[pl.ds(base, 16)], idx0s[b])
        pltpu.sync_copy(pos1_hbm.at[pl.ds(base, 16)], idx1s[b])
        g0 = pltpu.async_copy(os_hbm.at[idx0s[b]], r0s[b], semg[b])
        g1 = pltpu.async_copy(os_hbm.at[idx1s[b]], r1s[b], semg[b])
        gathers[c] = (g0, g1)

    start_chunk(0)
    for c in range(4):
        b = c % 2
        if c < 3:
            start_chunk(c + 1)
        gathers[c][0].wait()
        gathers[c][1].wait()
        if c >= 2:
            ywrites[c - 2].wait()
        ch0 = pr0_v[pl.ds(c * 16, 16)]
        ch1 = pr1_v[pl.ds(c * 16, 16)]
        r0b, r1b, yb = r0s[b], r1s[b], ys[b]

        def row(r, _, ch0=ch0, ch1=ch1, r0b=r0b, r1b=r1b, yb=yb):
            p0 = _bcast_lane(ch0, r)
            p1 = _bcast_lane(ch1, r)
            for cc in range(D_MODEL // 16):
                sl = pl.ds(cc * 16, 16)
                yb[r, sl] = p0 * r0b[r, sl] + p1 * r1b[r, sl]
            return 0

        lax.fori_loop(0, 16, row, 0)
        ywrites[c] = pltpu.async_copy(
            yb, y_hbm.at[pl.ds(tb + c * 16, 16), :], semy[b])
    ywrites[2].wait()
    ywrites[3].wait()


@functools.cache
def _make_combine():
    mesh = plsc.VectorSubcoreMesh(core_axis_name="c", subcore_axis_name="s")

    @functools.partial(
        pl.kernel,
        out_type=jax.ShapeDtypeStruct((T_TOK, D_MODEL), jnp.float32),
        mesh=mesh,
        scratch_types=[
            pltpu.VMEM((16,), jnp.int32),
            pltpu.VMEM((16,), jnp.int32),
            pltpu.VMEM((16,), jnp.int32),
            pltpu.VMEM((16,), jnp.int32),
            pltpu.VMEM((64,), jnp.float32),
            pltpu.VMEM((64,), jnp.float32),
            pltpu.VMEM((16, D_MODEL), jnp.float32),
            pltpu.VMEM((16, D_MODEL), jnp.float32),
            pltpu.VMEM((16, D_MODEL), jnp.float32),
            pltpu.VMEM((16, D_MODEL), jnp.float32),
            pltpu.VMEM((16, D_MODEL), jnp.float32),
            pltpu.VMEM((16, D_MODEL), jnp.float32),
            pltpu.SemaphoreType.DMA,
            pltpu.SemaphoreType.DMA,
            pltpu.SemaphoreType.DMA,
            pltpu.SemaphoreType.DMA,
        ],
    )
    def _combine(os_hbm, pos0_hbm, pos1_hbm, pr0_hbm, pr1_hbm, y_hbm,
                 idx00, idx01, idx10, idx11, pr0_v, pr1_v,
                 r00, r01, r10, r11, y0_v, y1_v,
                 semg0, semg1, semy0, semy1):
        _combine_body(os_hbm, pos0_hbm, pos1_hbm, pr0_hbm, pr1_hbm, y_hbm,
                      idx00, idx01, idx10, idx11, pr0_v, pr1_v,
                      r00, r01, r10, r11, y0_v, y1_v,
                      semg0, semg1, semy0, semy1)

    return _combine


def kernel(x, gate_w, w1, w2, w3):
    Bq, Tq, C = x.shape
    xf = x.reshape(Tq, C)
    pos, prob, te_blk = _route(xf, gate_w)
    te2d = te_blk[:2]

    pos_cat = jnp.concatenate([pos[:, 0], pos[:, 1]])
    xs = _make_dispatch()(xf, pos_cat)

    out_s = _grouped_mlp(te2d, xs, w1, w2, w3)

    y = _make_combine()(out_s, pos[:, 0], pos[:, 1], prob[:, 0], prob[:, 1])
    return y.reshape(Bq, Tq, C)
